# Initial kernel scaffold; baseline (speedup 1.0000x reference)
import functools

import jax
import jax.numpy as jnp
from jax import lax
from jax.experimental import pallas as pl
from jax.experimental.pallas import tpu as pltpu
from jax.experimental.pallas import tpu_sc as plsc

N = 100000
E = 6400000
NC = 2
NS = 16
IDXW = 125
K = 16
EROWS = E // IDXW
F = 16
ROWS_PER_TILE_OUT = N // NS


def _sc_mesh():
    return plsc.VectorSubcoreMesh(core_axis_name="c", subcore_axis_name="s")


def _make_agg_kernel(core_row_off, rows_per_tile):
    nchunks = rows_per_tile // K
    assert nchunks * K == rows_per_tile

    @functools.partial(
        pl.kernel,
        out_type=[
            jax.ShapeDtypeStruct((N, F), jnp.float32),
            jax.ShapeDtypeStruct((N, F), jnp.float32),
        ],
        mesh=_sc_mesh(),
        scratch_types=[
            pltpu.VMEM((K, IDXW), jnp.int32),
            pltpu.VMEM((K, IDXW), jnp.int32),
            pltpu.VMEM((K * IDXW, F), jnp.float32),
            pltpu.VMEM_SHARED((N, F), jnp.float32),
            pltpu.SemaphoreType.DMA,
            pltpu.SemaphoreType.DMA,
        ],
    )
    def agg(src2, dst2, ta, tb, zeros, outa, outb,
            idxv, dstv, rows, acc, gsem, ssem):
        cid = lax.axis_index("c")
        sid = lax.axis_index("s")

        pltpu.sync_copy(
            zeros.at[pl.ds(sid * ROWS_PER_TILE_OUT, ROWS_PER_TILE_OUT)],
            acc.at[pl.ds(sid * ROWS_PER_TILE_OUT, ROWS_PER_TILE_OUT)])
        plsc.subcore_barrier()

        def run(table):
            def chunk(i, carry):
                row0 = cid * core_row_off + sid * rows_per_tile + i * K
                pltpu.sync_copy(src2.at[pl.ds(row0, K)], idxv)
                pltpu.sync_copy(dst2.at[pl.ds(row0, K)], dstv)
                gds = [
                    pltpu.async_copy(
                        table.at[idxv.at[j]],
                        rows.at[pl.ds(j * IDXW, IDXW)], gsem)
                    for j in range(K)
                ]
                for d in gds:
                    d.wait()
                sds = [
                    pltpu.async_copy(
                        rows.at[pl.ds(j * IDXW, IDXW)],
                        acc.at[dstv.at[j]], ssem, add=True)
                    for j in range(K)
                ]
                for d in sds:
                    d.wait()
                return carry
            lax.fori_loop(0, nchunks, chunk, 0)

        @pl.when(cid == 0)
        def _():
            run(ta)

        @pl.when(cid == 1)
        def _():
            run(tb)

        plsc.subcore_barrier()

        sl = pl.ds(sid * ROWS_PER_TILE_OUT, ROWS_PER_TILE_OUT)

        @pl.when(cid == 0)
        def _():
            pltpu.sync_copy(acc.at[sl], outa.at[sl])

        @pl.when(cid == 1)
        def _():
            pltpu.sync_copy(acc.at[sl], outb.at[sl])

    return agg


def _make_deg_kernel():
    rows_per_tile = EROWS // (NC * NS)
    nchunks = rows_per_tile // K
    zrows = N // 4

    @functools.partial(
        pl.kernel,
        out_type=[
            jax.ShapeDtypeStruct((N, 1), jnp.float32),
            jax.ShapeDtypeStruct((N, 1), jnp.float32),
        ],
        mesh=_sc_mesh(),
        scratch_types=[
            pltpu.VMEM((K, IDXW), jnp.int32),
            pltpu.VMEM((IDXW, 1), jnp.float32),
            pltpu.VMEM_SHARED((N, 1), jnp.float32),
            pltpu.SemaphoreType.DMA,
        ],
    )
    def deg(dst2, ones, zeros1, outa, outb, dstv, ones_v, acc, ssem):
        cid = lax.axis_index("c")
        sid = lax.axis_index("s")

        pltpu.sync_copy(ones, ones_v)

        @pl.when(sid < 4)
        def _():
            pltpu.sync_copy(zeros1.at[pl.ds(sid * zrows, zrows)],
                            acc.at[pl.ds(sid * zrows, zrows)])
        plsc.subcore_barrier()

        def chunk(i, carry):
            row0 = (cid * NS + sid) * rows_per_tile + i * K
            pltpu.sync_copy(dst2.at[pl.ds(row0, K)], dstv)
            sds = [
                pltpu.async_copy(ones_v, acc.at[dstv.at[j]], ssem, add=True)
                for j in range(K)
            ]
            for d in sds:
                d.wait()
            return carry
        lax.fori_loop(0, nchunks, chunk, 0)

        plsc.subcore_barrier()

        @pl.when(jnp.logical_and(cid == 0, sid < 4))
        def _():
            sl = pl.ds(sid * zrows, zrows)
            pltpu.sync_copy(acc.at[sl], outa.at[sl])

        @pl.when(jnp.logical_and(cid == 1, sid < 4))
        def _():
            sl = pl.ds(sid * zrows, zrows)
            pltpu.sync_copy(acc.at[sl], outb.at[sl])

    return deg


_B = 1000


def _row_spec(w):
    return pl.BlockSpec((_B, w), lambda i: (i, 0))


def _full_spec(shape):
    return pl.BlockSpec(shape, lambda i: (0, 0))


def _tc_a_body(x_r, w1_r, da_r, db_r, u1a_r, u1b_r, dis_r):
    deg = da_r[...] + db_r[...] + 1.0
    dis = lax.rsqrt(deg)
    dis_r[...] = dis
    xw = jnp.dot(x_r[...], w1_r[...], preferred_element_type=jnp.float32)
    u = xw * dis
    u1a_r[...] = u[:, :16]
    u1b_r[...] = u[:, 16:]


def _tc_b_body(aa_r, ab_r, ua_r, ub_r, dis_r, w2a_r, w2b_r, b1_r, u2_r):
    dis = dis_r[...]
    ha = jnp.maximum((aa_r[...] + ua_r[...]) * dis + b1_r[...][:, :16], 0.0)
    hb = jnp.maximum((ab_r[...] + ub_r[...]) * dis + b1_r[...][:, 16:], 0.0)
    t = (jnp.dot(ha, w2a_r[...], preferred_element_type=jnp.float32)
         + jnp.dot(hb, w2b_r[...], preferred_element_type=jnp.float32))
    u2_r[...] = t * dis


def _tc_c_body(aa_r, ab_r, u2_r, dis_r, b2_r, wfc_r, bfc_r, out_r):
    h = jnp.maximum((aa_r[...] + ab_r[...] + u2_r[...]) * dis_r[...]
                    + b2_r[...], 0.0)
    out_r[...] = (jnp.dot(h, wfc_r[...], preferred_element_type=jnp.float32)
                  + bfc_r[...])


def kernel(x, edge_index, W1, b1, W2, b2, Wfc, bfc):
    src2 = edge_index[0].reshape(EROWS, IDXW)
    dst2 = edge_index[1].reshape(EROWS, IDXW)
    xp = jnp.pad(x, ((0, 0), (0, 1)))
    W1p = jnp.pad(W1, ((0, 1), (0, 0)))
    zeros16 = jnp.zeros((N, F), jnp.float32)
    zeros1 = jnp.zeros((N, 1), jnp.float32)
    ones = jnp.ones((IDXW, 1), jnp.float32)

    dega, degb = _make_deg_kernel()(dst2, ones, zeros1)

    u1a, u1b, dis = pl.pallas_call(
        _tc_a_body,
        grid=(N // _B,),
        in_specs=[_row_spec(8), _full_spec((8, 32)), _row_spec(1), _row_spec(1)],
        out_specs=[_row_spec(16), _row_spec(16), _row_spec(1)],
        out_shape=[
            jax.ShapeDtypeStruct((N, 16), jnp.float32),
            jax.ShapeDtypeStruct((N, 16), jnp.float32),
            jax.ShapeDtypeStruct((N, 1), jnp.float32),
        ],
    )(xp, W1p, dega, degb)

    agg1a, agg1b = _make_agg_kernel(0, EROWS // NS)(
        src2, dst2, u1a, u1b, zeros16)

    u2 = pl.pallas_call(
        _tc_b_body,
        grid=(N // _B,),
        in_specs=[_row_spec(16), _row_spec(16), _row_spec(16), _row_spec(16),
                  _row_spec(1), _full_spec((16, 16)), _full_spec((16, 16)),
                  _full_spec((1, 32))],
        out_specs=_row_spec(16),
        out_shape=jax.ShapeDtypeStruct((N, 16), jnp.float32),
    )(agg1a, agg1b, u1a, u1b, dis, W2[:16], W2[16:], b1.reshape(1, 32))

    agg2a, agg2b = _make_agg_kernel(EROWS // NC, EROWS // (NC * NS))(
        src2, dst2, u2, u2, zeros16)

    out = pl.pallas_call(
        _tc_c_body,
        grid=(N // _B,),
        in_specs=[_row_spec(16), _row_spec(16), _row_spec(16), _row_spec(1),
                  _full_spec((1, 16)), _full_spec((16, 2)), _full_spec((1, 2))],
        out_specs=_row_spec(2),
        out_shape=jax.ShapeDtypeStruct((N, 2), jnp.float32),
    )(agg2a, agg2b, u2, dis, b2.reshape(1, 16), Wfc, bfc.reshape(1, 2))

    return out


# speedup vs baseline: 48.4557x; 48.4557x over previous
import functools

import jax
import jax.numpy as jnp
from jax import lax
from jax.experimental import pallas as pl
from jax.experimental.pallas import tpu as pltpu
from jax.experimental.pallas import tpu_sc as plsc

N = 100000
E = 6400000
NC = 2
NS = 16
IDXW = 125
K = 10
EROWS = E // IDXW
F = 16
ZC = 4000
NZC = N // ZC


def _acc_chunks(sid, fn):
    fn(sid * ZC)

    @pl.when(sid < NZC - NS)
    def _():
        fn((sid + NS) * ZC)


def _sc_mesh():
    return plsc.VectorSubcoreMesh(core_axis_name="c", subcore_axis_name="s")


def _make_agg_kernel(core_row_off, rows_per_tile):
    nchunks = rows_per_tile // K
    assert nchunks * K == rows_per_tile

    @functools.partial(
        pl.kernel,
        out_type=[
            jax.ShapeDtypeStruct((N, F), jnp.float32),
            jax.ShapeDtypeStruct((N, F), jnp.float32),
        ],
        mesh=_sc_mesh(),
        scratch_types=[
            pltpu.VMEM((K, IDXW), jnp.int32),
            pltpu.VMEM((K, IDXW), jnp.int32),
            pltpu.VMEM((K, IDXW, F), jnp.float32),
            pltpu.VMEM_SHARED((N, F), jnp.float32),
            pltpu.SemaphoreType.DMA,
            pltpu.SemaphoreType.DMA,
        ],
        compiler_params=pltpu.CompilerParams(use_tc_tiling_on_sc=False),
    )
    def agg(src2, dst2, ta, tb, zeros, outa, outb,
            idxv, dstv, rows, acc, gsem, ssem):
        cid = lax.axis_index("c")
        sid = lax.axis_index("s")

        _acc_chunks(sid, lambda r0: pltpu.sync_copy(
            zeros.at[pl.ds(r0, ZC)], acc.at[pl.ds(r0, ZC)]))
        plsc.subcore_barrier()

        def run(table):
            def chunk(i, carry):
                row0 = cid * core_row_off + sid * rows_per_tile + i * K
                pltpu.sync_copy(src2.at[pl.ds(row0, K)], idxv)
                pltpu.sync_copy(dst2.at[pl.ds(row0, K)], dstv)
                gds = [
                    pltpu.async_copy(
                        table.at[idxv.at[j]], rows.at[j], gsem)
                    for j in range(K)
                ]
                for d in gds:
                    d.wait()
                sds = [
                    pltpu.async_copy(
                        rows.at[j], acc.at[dstv.at[j]], ssem, add=True)
                    for j in range(K)
                ]
                for d in sds:
                    d.wait()
                return carry
            lax.fori_loop(0, nchunks, chunk, 0)

        @pl.when(cid == 0)
        def _():
            run(ta)

        @pl.when(cid == 1)
        def _():
            run(tb)

        plsc.subcore_barrier()

        @pl.when(cid == 0)
        def _():
            _acc_chunks(sid, lambda r0: pltpu.sync_copy(
                acc.at[pl.ds(r0, ZC)], outa.at[pl.ds(r0, ZC)]))

        @pl.when(cid == 1)
        def _():
            _acc_chunks(sid, lambda r0: pltpu.sync_copy(
                acc.at[pl.ds(r0, ZC)], outb.at[pl.ds(r0, ZC)]))

    return agg


def _make_deg_kernel():
    rows_per_tile = EROWS // (NC * NS)
    nchunks = rows_per_tile // K

    @functools.partial(
        pl.kernel,
        out_type=[
            jax.ShapeDtypeStruct((N, F), jnp.float32),
            jax.ShapeDtypeStruct((N, F), jnp.float32),
        ],
        mesh=_sc_mesh(),
        scratch_types=[
            pltpu.VMEM((K, IDXW), jnp.int32),
            pltpu.VMEM((IDXW, F), jnp.float32),
            pltpu.VMEM_SHARED((N, F), jnp.float32),
            pltpu.SemaphoreType.DMA,
        ],
        compiler_params=pltpu.CompilerParams(use_tc_tiling_on_sc=False),
    )
    def deg(dst2, ones, zeros, outa, outb, dstv, ones_v, acc, ssem):
        cid = lax.axis_index("c")
        sid = lax.axis_index("s")

        pltpu.sync_copy(ones, ones_v)
        _acc_chunks(sid, lambda r0: pltpu.sync_copy(
            zeros.at[pl.ds(r0, ZC)], acc.at[pl.ds(r0, ZC)]))
        plsc.subcore_barrier()

        def chunk(i, carry):
            row0 = (cid * NS + sid) * rows_per_tile + i * K
            pltpu.sync_copy(dst2.at[pl.ds(row0, K)], dstv)
            sds = [
                pltpu.async_copy(ones_v, acc.at[dstv.at[j]], ssem, add=True)
                for j in range(K)
            ]
            for d in sds:
                d.wait()
            return carry
        lax.fori_loop(0, nchunks, chunk, 0)

        plsc.subcore_barrier()

        @pl.when(cid == 0)
        def _():
            _acc_chunks(sid, lambda r0: pltpu.sync_copy(
                acc.at[pl.ds(r0, ZC)], outa.at[pl.ds(r0, ZC)]))

        @pl.when(cid == 1)
        def _():
            _acc_chunks(sid, lambda r0: pltpu.sync_copy(
                acc.at[pl.ds(r0, ZC)], outb.at[pl.ds(r0, ZC)]))

    return deg


_B = 1000


def _row_spec(w):
    return pl.BlockSpec((_B, w), lambda i: (i, 0))


def _full_spec(shape):
    return pl.BlockSpec(shape, lambda i: (0, 0))


def _tc_a_body(x_r, w1_r, da_r, db_r, u1a_r, u1b_r, dis_r):
    deg = da_r[:, 0:1] + db_r[:, 0:1] + 1.0
    dis = lax.rsqrt(deg)
    dis_r[...] = dis
    xw = jnp.dot(x_r[...], w1_r[...], preferred_element_type=jnp.float32)
    u = xw * dis
    u1a_r[...] = u[:, :16]
    u1b_r[...] = u[:, 16:]


def _tc_b_body(aa_r, ab_r, ua_r, ub_r, dis_r, w2a_r, w2b_r, b1_r, u2_r):
    dis = dis_r[...]
    ha = jnp.maximum((aa_r[...] + ua_r[...]) * dis + b1_r[...][:, :16], 0.0)
    hb = jnp.maximum((ab_r[...] + ub_r[...]) * dis + b1_r[...][:, 16:], 0.0)
    t = (jnp.dot(ha, w2a_r[...], preferred_element_type=jnp.float32)
         + jnp.dot(hb, w2b_r[...], preferred_element_type=jnp.float32))
    u2_r[...] = t * dis


def _tc_c_body(aa_r, ab_r, u2_r, dis_r, b2_r, wfc_r, bfc_r, out_r):
    h = jnp.maximum((aa_r[...] + ab_r[...] + u2_r[...]) * dis_r[...]
                    + b2_r[...], 0.0)
    out_r[...] = (jnp.dot(h, wfc_r[...], preferred_element_type=jnp.float32)
                  + bfc_r[...])


def kernel(x, edge_index, W1, b1, W2, b2, Wfc, bfc):
    src2 = edge_index[0].reshape(EROWS, IDXW)
    dst2 = edge_index[1].reshape(EROWS, IDXW)
    xp = jnp.pad(x, ((0, 0), (0, 1)))
    W1p = jnp.pad(W1, ((0, 1), (0, 0)))
    zeros16 = jnp.zeros((N, F), jnp.float32)
    ones = jnp.ones((IDXW, F), jnp.float32)

    dega, degb = _make_deg_kernel()(dst2, ones, zeros16)

    u1a, u1b, dis = pl.pallas_call(
        _tc_a_body,
        grid=(N // _B,),
        in_specs=[_row_spec(8), _full_spec((8, 32)), _row_spec(16), _row_spec(16)],
        out_specs=[_row_spec(16), _row_spec(16), _row_spec(1)],
        out_shape=[
            jax.ShapeDtypeStruct((N, 16), jnp.float32),
            jax.ShapeDtypeStruct((N, 16), jnp.float32),
            jax.ShapeDtypeStruct((N, 1), jnp.float32),
        ],
    )(xp, W1p, dega, degb)

    agg1a, agg1b = _make_agg_kernel(0, EROWS // NS)(
        src2, dst2, u1a, u1b, zeros16)

    u2 = pl.pallas_call(
        _tc_b_body,
        grid=(N // _B,),
        in_specs=[_row_spec(16), _row_spec(16), _row_spec(16), _row_spec(16),
                  _row_spec(1), _full_spec((16, 16)), _full_spec((16, 16)),
                  _full_spec((1, 32))],
        out_specs=_row_spec(16),
        out_shape=jax.ShapeDtypeStruct((N, 16), jnp.float32),
    )(agg1a, agg1b, u1a, u1b, dis, W2[:16], W2[16:], b1.reshape(1, 32))

    agg2a, agg2b = _make_agg_kernel(EROWS // NC, EROWS // (NC * NS))(
        src2, dst2, u2, u2, zeros16)

    out = pl.pallas_call(
        _tc_c_body,
        grid=(N // _B,),
        in_specs=[_row_spec(16), _row_spec(16), _row_spec(16), _row_spec(1),
                  _full_spec((1, 16)), _full_spec((16, 2)), _full_spec((1, 2))],
        out_specs=_row_spec(2),
        out_shape=jax.ShapeDtypeStruct((N, 2), jnp.float32),
    )(agg2a, agg2b, u2, dis, b2.reshape(1, 16), Wfc, bfc.reshape(1, 2))

    return out


# speedup vs baseline: 58.5668x; 1.2087x over previous
import functools

import jax
import jax.numpy as jnp
from jax import lax
from jax.experimental import pallas as pl
from jax.experimental.pallas import tpu as pltpu
from jax.experimental.pallas import tpu_sc as plsc

N = 100000
E = 6400000
NC = 2
NS = 16
IDXW = 125
K = 5
EROWS = E // IDXW
F = 16
ZC = 4000
NZC = N // ZC


def _acc_chunks(sid, fn):
    fn(sid * ZC)

    @pl.when(sid < NZC - NS)
    def _():
        fn((sid + NS) * ZC)


def _sc_mesh():
    return plsc.VectorSubcoreMesh(core_axis_name="c", subcore_axis_name="s")


def _make_agg_kernel(core_row_off, rows_per_tile):
    nchunks = rows_per_tile // K
    assert nchunks * K == rows_per_tile and nchunks % 4 == 0 and nchunks >= 8
    CB = K * IDXW

    @functools.partial(
        pl.kernel,
        out_type=[
            jax.ShapeDtypeStruct((N, F), jnp.float32),
            jax.ShapeDtypeStruct((N, F), jnp.float32),
        ],
        mesh=_sc_mesh(),
        scratch_types=[
            pltpu.VMEM((2, K, IDXW), jnp.int32),
            pltpu.VMEM((4, K, IDXW), jnp.int32),
            pltpu.VMEM((2, CB, F), jnp.float32),
            pltpu.VMEM_SHARED((N, F), jnp.float32),
            pltpu.SemaphoreType.DMA,
            pltpu.SemaphoreType.DMA,
            pltpu.SemaphoreType.DMA,
        ],
        compiler_params=pltpu.CompilerParams(use_tc_tiling_on_sc=False),
    )
    def agg(src2, dst2, ta, tb, zeros, outa, outb,
            idxv, dstv, rows, acc, isem, gsem, ssem):
        cid = lax.axis_index("c")
        sid = lax.axis_index("s")

        _acc_chunks(sid, lambda r0: pltpu.sync_copy(
            zeros.at[pl.ds(r0, ZC)], acc.at[pl.ds(r0, ZC)]))
        plsc.subcore_barrier()

        def run(table):
            base = cid * core_row_off + sid * rows_per_tile

            def load(i, islot, dslot):
                r0 = base + i * K
                pltpu.async_copy(src2.at[pl.ds(r0, K)], idxv.at[islot], isem)
                pltpu.async_copy(dst2.at[pl.ds(r0, K)], dstv.at[dslot], isem)

            def drain_loads(islot):
                pltpu.make_async_copy(
                    src2.at[pl.ds(base, K)], idxv.at[islot], isem).wait()
                pltpu.make_async_copy(
                    src2.at[pl.ds(base, K)], idxv.at[islot], isem).wait()

            def drain_scatters():
                pltpu.make_async_copy(
                    zeros.at[pl.ds(0, CB)], rows.at[0], ssem).wait()

            load(0, 0, 0)

            def chunk4(i4, carry):
                for b in range(4):
                    i = i4 * 4 + b
                    rs = b % 2

                    @pl.when(i >= 2)
                    def _():
                        drain_scatters()

                    drain_loads(rs)
                    gds = [
                        pltpu.async_copy(
                            table.at[idxv.at[rs, j]],
                            rows.at[rs, pl.ds(j * IDXW, IDXW)], gsem)
                        for j in range(K)
                    ]

                    @pl.when(i + 1 < nchunks)
                    def _():
                        load(i + 1, 1 - rs, (b + 1) % 4)

                    for d in gds:
                        d.wait()
                    for j in range(K):
                        pltpu.async_copy(
                            rows.at[rs, pl.ds(j * IDXW, IDXW)],
                            acc.at[dstv.at[b, j]], ssem, add=True)
                return carry
            lax.fori_loop(0, nchunks // 4, chunk4, 0)
            drain_scatters()
            drain_scatters()

        @pl.when(cid == 0)
        def _():
            run(ta)

        @pl.when(cid == 1)
        def _():
            run(tb)

        plsc.subcore_barrier()

        @pl.when(cid == 0)
        def _():
            _acc_chunks(sid, lambda r0: pltpu.sync_copy(
                acc.at[pl.ds(r0, ZC)], outa.at[pl.ds(r0, ZC)]))

        @pl.when(cid == 1)
        def _():
            _acc_chunks(sid, lambda r0: pltpu.sync_copy(
                acc.at[pl.ds(r0, ZC)], outb.at[pl.ds(r0, ZC)]))

    return agg


def _make_deg_kernel():
    rows_per_tile = EROWS // (NC * NS)
    nchunks = rows_per_tile // K

    @functools.partial(
        pl.kernel,
        out_type=[
            jax.ShapeDtypeStruct((N, F), jnp.float32),
            jax.ShapeDtypeStruct((N, F), jnp.float32),
        ],
        mesh=_sc_mesh(),
        scratch_types=[
            pltpu.VMEM((K, IDXW), jnp.int32),
            pltpu.VMEM((IDXW, F), jnp.float32),
            pltpu.VMEM_SHARED((N, F), jnp.float32),
            pltpu.SemaphoreType.DMA,
        ],
        compiler_params=pltpu.CompilerParams(use_tc_tiling_on_sc=False),
    )
    def deg(dst2, ones, zeros, outa, outb, dstv, ones_v, acc, ssem):
        cid = lax.axis_index("c")
        sid = lax.axis_index("s")

        pltpu.sync_copy(ones, ones_v)
        _acc_chunks(sid, lambda r0: pltpu.sync_copy(
            zeros.at[pl.ds(r0, ZC)], acc.at[pl.ds(r0, ZC)]))
        plsc.subcore_barrier()

        def chunk(i, carry):
            row0 = (cid * NS + sid) * rows_per_tile + i * K
            pltpu.sync_copy(dst2.at[pl.ds(row0, K)], dstv)
            sds = [
                pltpu.async_copy(ones_v, acc.at[dstv.at[j]], ssem, add=True)
                for j in range(K)
            ]
            for d in sds:
                d.wait()
            return carry
        lax.fori_loop(0, nchunks, chunk, 0)

        plsc.subcore_barrier()

        @pl.when(cid == 0)
        def _():
            _acc_chunks(sid, lambda r0: pltpu.sync_copy(
                acc.at[pl.ds(r0, ZC)], outa.at[pl.ds(r0, ZC)]))

        @pl.when(cid == 1)
        def _():
            _acc_chunks(sid, lambda r0: pltpu.sync_copy(
                acc.at[pl.ds(r0, ZC)], outb.at[pl.ds(r0, ZC)]))

    return deg


_B = 1000


def _row_spec(w):
    return pl.BlockSpec((_B, w), lambda i: (i, 0))


def _full_spec(shape):
    return pl.BlockSpec(shape, lambda i: (0, 0))


def _tc_a_body(x_r, w1_r, da_r, db_r, u1a_r, u1b_r, dis_r):
    deg = da_r[:, 0:1] + db_r[:, 0:1] + 1.0
    dis = lax.rsqrt(deg)
    dis_r[...] = dis
    xw = jnp.dot(x_r[...], w1_r[...], preferred_element_type=jnp.float32)
    u = xw * dis
    u1a_r[...] = u[:, :16]
    u1b_r[...] = u[:, 16:]


def _tc_b_body(aa_r, ab_r, ua_r, ub_r, dis_r, w2a_r, w2b_r, b1_r, u2_r):
    dis = dis_r[...]
    ha = jnp.maximum((aa_r[...] + ua_r[...]) * dis + b1_r[...][:, :16], 0.0)
    hb = jnp.maximum((ab_r[...] + ub_r[...]) * dis + b1_r[...][:, 16:], 0.0)
    t = (jnp.dot(ha, w2a_r[...], preferred_element_type=jnp.float32)
         + jnp.dot(hb, w2b_r[...], preferred_element_type=jnp.float32))
    u2_r[...] = t * dis


def _tc_c_body(aa_r, ab_r, u2_r, dis_r, b2_r, wfc_r, bfc_r, out_r):
    h = jnp.maximum((aa_r[...] + ab_r[...] + u2_r[...]) * dis_r[...]
                    + b2_r[...], 0.0)
    out_r[...] = (jnp.dot(h, wfc_r[...], preferred_element_type=jnp.float32)
                  + bfc_r[...])


def kernel(x, edge_index, W1, b1, W2, b2, Wfc, bfc):
    src2 = edge_index[0].reshape(EROWS, IDXW)
    dst2 = edge_index[1].reshape(EROWS, IDXW)
    xp = jnp.pad(x, ((0, 0), (0, 1)))
    W1p = jnp.pad(W1, ((0, 1), (0, 0)))
    zeros16 = jnp.zeros((N, F), jnp.float32)
    ones = jnp.ones((IDXW, F), jnp.float32)

    dega, degb = _make_deg_kernel()(dst2, ones, zeros16)

    u1a, u1b, dis = pl.pallas_call(
        _tc_a_body,
        grid=(N // _B,),
        in_specs=[_row_spec(8), _full_spec((8, 32)), _row_spec(16), _row_spec(16)],
        out_specs=[_row_spec(16), _row_spec(16), _row_spec(1)],
        out_shape=[
            jax.ShapeDtypeStruct((N, 16), jnp.float32),
            jax.ShapeDtypeStruct((N, 16), jnp.float32),
            jax.ShapeDtypeStruct((N, 1), jnp.float32),
        ],
    )(xp, W1p, dega, degb)

    agg1a, agg1b = _make_agg_kernel(0, EROWS // NS)(
        src2, dst2, u1a, u1b, zeros16)

    u2 = pl.pallas_call(
        _tc_b_body,
        grid=(N // _B,),
        in_specs=[_row_spec(16), _row_spec(16), _row_spec(16), _row_spec(16),
                  _row_spec(1), _full_spec((16, 16)), _full_spec((16, 16)),
                  _full_spec((1, 32))],
        out_specs=_row_spec(16),
        out_shape=jax.ShapeDtypeStruct((N, 16), jnp.float32),
    )(agg1a, agg1b, u1a, u1b, dis, W2[:16], W2[16:], b1.reshape(1, 32))

    agg2a, agg2b = _make_agg_kernel(EROWS // NC, EROWS // (NC * NS))(
        src2, dst2, u2, u2, zeros16)

    out = pl.pallas_call(
        _tc_c_body,
        grid=(N // _B,),
        in_specs=[_row_spec(16), _row_spec(16), _row_spec(16), _row_spec(1),
                  _full_spec((1, 16)), _full_spec((16, 2)), _full_spec((1, 2))],
        out_specs=_row_spec(2),
        out_shape=jax.ShapeDtypeStruct((N, 2), jnp.float32),
    )(agg2a, agg2b, u2, dis, b2.reshape(1, 16), Wfc, bfc.reshape(1, 2))

    return out


# speedup vs baseline: 96.1974x; 1.6425x over previous
import functools

import jax
import jax.numpy as jnp
from jax import lax
from jax.experimental import pallas as pl
from jax.experimental.pallas import tpu as pltpu
from jax.experimental.pallas import tpu_sc as plsc

N = 100000
E = 6400000
NC = 2
NS = 16
IDXW = 128
G = E // IDXW
K = 5
CB = K * IDXW
F = 16
ZC = 4000
NZC = N // ZC
R = N // 8

C1_MAIN = 624
C2_MAIN = 312
TAIL_G0 = 49920


def _sc_mesh():
    return plsc.VectorSubcoreMesh(core_axis_name="c", subcore_axis_name="s")


def _acc_chunks(sid, fn):
    fn(sid * ZC)

    @pl.when(sid < NZC - NS)
    def _():
        fn((sid + NS) * ZC)


def _make_agg_kernel(feature_split):
    main_cpt = C1_MAIN if feature_split else C2_MAIN

    @functools.partial(
        pl.kernel,
        out_type=[
            jax.ShapeDtypeStruct((N, F), jnp.float32),
            jax.ShapeDtypeStruct((N, F), jnp.float32),
        ],
        mesh=_sc_mesh(),
        scratch_types=[
            pltpu.VMEM((2, K, IDXW), jnp.int32),
            pltpu.VMEM((4, K, IDXW), jnp.int32),
            pltpu.VMEM((2, CB, F), jnp.float32),
            pltpu.VMEM_SHARED((N, F), jnp.float32),
            pltpu.SemaphoreType.DMA,
            pltpu.SemaphoreType.DMA,
            pltpu.SemaphoreType.DMA,
        ],
        compiler_params=pltpu.CompilerParams(use_tc_tiling_on_sc=False),
    )
    def agg(ei3, ta, tb, zeros, outa, outb,
            idxv, dstv, rows, acc, isem, gsem, ssem):
        cid = lax.axis_index("c")
        sid = lax.axis_index("s")

        _acc_chunks(sid, lambda r0: pltpu.sync_copy(
            zeros.at[pl.ds(r0, ZC)], acc.at[pl.ds(r0, ZC)]))
        plsc.subcore_barrier()

        def run(table):
            if feature_split:
                base_g = sid * (C1_MAIN * K)
                tail_g = TAIL_G0 + sid * K
                tail_on = sid >= 0
            else:
                wid = cid * NS + sid
                base_g = wid * (C2_MAIN * K)
                tail_g = TAIL_G0 + wid * K
                tail_on = wid < NS

            def load(i, islot, dslot):
                g0 = base_g + i * K
                pltpu.async_copy(ei3.at[pl.ds(g0, K), 0], idxv.at[islot], isem)
                pltpu.async_copy(ei3.at[pl.ds(g0, K), 1], dstv.at[dslot], isem)

            def drain_loads(islot):
                for _ in range(2):
                    pltpu.make_async_copy(
                        ei3.at[pl.ds(0, K), 0], idxv.at[islot], isem).wait()

            def drain_scatters():
                pltpu.make_async_copy(
                    zeros.at[pl.ds(0, CB)], rows.at[0], ssem).wait()

            load(0, 0, 0)

            def chunk4(i4, carry):
                for b in range(4):
                    i = i4 * 4 + b
                    rs = b % 2

                    @pl.when(i >= 2)
                    def _():
                        drain_scatters()

                    drain_loads(rs)
                    gds = [
                        pltpu.async_copy(
                            table.at[idxv.at[rs, j]],
                            rows.at[rs, pl.ds(j * IDXW, IDXW)], gsem)
                        for j in range(K)
                    ]

                    @pl.when(i + 1 < main_cpt)
                    def _():
                        load(i + 1, 1 - rs, (b + 1) % 4)

                    for d in gds:
                        d.wait()
                    for j in range(K):
                        pltpu.async_copy(
                            rows.at[rs, pl.ds(j * IDXW, IDXW)],
                            acc.at[dstv.at[b, j]], ssem, add=True)
                return carry
            lax.fori_loop(0, main_cpt // 4, chunk4, 0)
            drain_scatters()
            drain_scatters()

            @pl.when(tail_on)
            def _():
                pltpu.sync_copy(ei3.at[pl.ds(tail_g, K), 0], idxv.at[0])
                pltpu.sync_copy(ei3.at[pl.ds(tail_g, K), 1], dstv.at[0])
                gds = [
                    pltpu.async_copy(
                        table.at[idxv.at[0, j]],
                        rows.at[0, pl.ds(j * IDXW, IDXW)], gsem)
                    for j in range(K)
                ]
                for d in gds:
                    d.wait()
                sds = [
                    pltpu.async_copy(
                        rows.at[0, pl.ds(j * IDXW, IDXW)],
                        acc.at[dstv.at[0, j]], ssem, add=True)
                    for j in range(K)
                ]
                for d in sds:
                    d.wait()

        @pl.when(cid == 0)
        def _():
            run(ta)

        @pl.when(cid == 1)
        def _():
            run(tb)

        plsc.subcore_barrier()

        @pl.when(cid == 0)
        def _():
            _acc_chunks(sid, lambda r0: pltpu.sync_copy(
                acc.at[pl.ds(r0, ZC)], outa.at[pl.ds(r0, ZC)]))

        @pl.when(cid == 1)
        def _():
            _acc_chunks(sid, lambda r0: pltpu.sync_copy(
                acc.at[pl.ds(r0, ZC)], outb.at[pl.ds(r0, ZC)]))

    return agg


def _make_deg_kernel():
    @functools.partial(
        pl.kernel,
        out_type=[
            jax.ShapeDtypeStruct((N, F), jnp.float32),
            jax.ShapeDtypeStruct((N, F), jnp.float32),
        ],
        mesh=_sc_mesh(),
        scratch_types=[
            pltpu.VMEM((4, K, IDXW), jnp.int32),
            pltpu.VMEM((IDXW, F), jnp.float32),
            pltpu.VMEM_SHARED((N, F), jnp.float32),
            pltpu.SemaphoreType.DMA,
            pltpu.SemaphoreType.DMA,
        ],
        compiler_params=pltpu.CompilerParams(use_tc_tiling_on_sc=False),
    )
    def deg(ei3, ones, zeros, outa, outb, dstv, ones_v, acc, isem, ssem):
        cid = lax.axis_index("c")
        sid = lax.axis_index("s")
        wid = cid * NS + sid
        base_g = wid * (C2_MAIN * K)

        pltpu.sync_copy(ones, ones_v)
        _acc_chunks(sid, lambda r0: pltpu.sync_copy(
            zeros.at[pl.ds(r0, ZC)], acc.at[pl.ds(r0, ZC)]))
        plsc.subcore_barrier()

        def load(i, dslot):
            pltpu.async_copy(
                ei3.at[pl.ds(base_g + i * K, K), 1], dstv.at[dslot], isem)

        def drain_load(dslot):
            pltpu.make_async_copy(
                ei3.at[pl.ds(0, K), 1], dstv.at[dslot], isem).wait()

        def drain_scatters():
            for _ in range(K):
                pltpu.make_async_copy(
                    zeros.at[pl.ds(0, IDXW)], ones_v, ssem).wait()

        load(0, 0)

        def chunk4(i4, carry):
            for b in range(4):
                i = i4 * 4 + b

                @pl.when(i >= 2)
                def _():
                    drain_scatters()

                drain_load(b)

                @pl.when(i + 1 < C2_MAIN)
                def _():
                    load(i + 1, (b + 1) % 4)

                for j in range(K):
                    pltpu.async_copy(
                        ones_v, acc.at[dstv.at[b, j]], ssem, add=True)
            return carry
        lax.fori_loop(0, C2_MAIN // 4, chunk4, 0)
        drain_scatters()
        drain_scatters()

        @pl.when(wid < NS)
        def _():
            pltpu.sync_copy(
                ei3.at[pl.ds(TAIL_G0 + wid * K, K), 1], dstv.at[0])
            sds = [
                pltpu.async_copy(ones_v, acc.at[dstv.at[0, j]], ssem, add=True)
                for j in range(K)
            ]
            for d in sds:
                d.wait()

        plsc.subcore_barrier()

        @pl.when(cid == 0)
        def _():
            _acc_chunks(sid, lambda r0: pltpu.sync_copy(
                acc.at[pl.ds(r0, ZC)], outa.at[pl.ds(r0, ZC)]))

        @pl.when(cid == 1)
        def _():
            _acc_chunks(sid, lambda r0: pltpu.sync_copy(
                acc.at[pl.ds(r0, ZC)], outb.at[pl.ds(r0, ZC)]))

    return deg


_BR = 1256


def _blk_spec(w=128):
    return pl.BlockSpec((_BR, w), lambda i: (i, 0))


def _full_spec(shape):
    return pl.BlockSpec(shape, lambda i: (0, 0))


def _tc_a_body(x_r, m1a_r, m1b_r, da_r, db_r, u1a_r, u1b_r, dis_r):
    dis = lax.rsqrt(da_r[...] + db_r[...] + 1.0)
    dis_r[...] = dis
    x = x_r[...]
    u1a_r[...] = jnp.dot(x, m1a_r[...], preferred_element_type=jnp.float32) * dis
    u1b_r[...] = jnp.dot(x, m1b_r[...], preferred_element_type=jnp.float32) * dis


def _tc_b_body(aa_r, ab_r, ua_r, ub_r, dis_r, m2a_r, m2b_r, b1a_r, b1b_r, u2_r):
    dis = dis_r[...]
    ha = jnp.maximum((aa_r[...] + ua_r[...]) * dis + b1a_r[...], 0.0)
    hb = jnp.maximum((ab_r[...] + ub_r[...]) * dis + b1b_r[...], 0.0)
    t = (jnp.dot(ha, m2a_r[...], preferred_element_type=jnp.float32)
         + jnp.dot(hb, m2b_r[...], preferred_element_type=jnp.float32))
    u2_r[...] = t * dis


def _tc_c_body(aa_r, ab_r, u2_r, dis_r, mfc_r, b2_r, bfc_r, out_r):
    h = jnp.maximum((aa_r[...] + ab_r[...] + u2_r[...]) * dis_r[...]
                    + b2_r[...], 0.0)
    out_r[...] = (jnp.dot(h, mfc_r[...], preferred_element_type=jnp.float32)
                  + bfc_r[...])


def kernel(x, edge_index, W1, b1, W2, b2, Wfc, bfc):
    f32 = jnp.float32
    ei3 = jnp.transpose(edge_index.reshape(2, G, IDXW), (1, 0, 2))
    xb = jnp.pad(x, ((0, 0), (0, 1))).reshape(R, 64)
    eye8 = jnp.eye(8, dtype=f32)
    W1p = jnp.pad(W1, ((0, 1), (0, 0)))
    M1a = jnp.kron(eye8, W1p[:, :16])
    M1b = jnp.kron(eye8, W1p[:, 16:])
    M2a = jnp.kron(eye8, W2[:16])
    M2b = jnp.kron(eye8, W2[16:])
    Mfc = jnp.kron(eye8, Wfc)
    b1a_t = jnp.tile(b1[:16], 8).reshape(1, 128)
    b1b_t = jnp.tile(b1[16:], 8).reshape(1, 128)
    b2_t = jnp.tile(b2, 8).reshape(1, 128)
    bfc_t = jnp.tile(bfc, 8).reshape(1, 16)
    zeros16 = jnp.zeros((N, F), f32)
    ones = jnp.ones((IDXW, F), f32)

    dega, degb = _make_deg_kernel()(ei3, ones, zeros16)

    grid = (R + _BR - 1) // _BR
    u1a_b, u1b_b, dis_b = pl.pallas_call(
        _tc_a_body,
        grid=(grid,),
        in_specs=[_blk_spec(64), _full_spec((64, 128)), _full_spec((64, 128)),
                  _blk_spec(), _blk_spec()],
        out_specs=[_blk_spec(), _blk_spec(), _blk_spec()],
        out_shape=[jax.ShapeDtypeStruct((R, 128), f32)] * 3,
    )(xb, M1a, M1b, dega.reshape(R, 128), degb.reshape(R, 128))

    agg1a, agg1b = _make_agg_kernel(True)(
        ei3, u1a_b.reshape(N, F), u1b_b.reshape(N, F), zeros16)

    u2_b = pl.pallas_call(
        _tc_b_body,
        grid=(grid,),
        in_specs=[_blk_spec(), _blk_spec(), _blk_spec(), _blk_spec(),
                  _blk_spec(), _full_spec((128, 128)), _full_spec((128, 128)),
                  _full_spec((1, 128)), _full_spec((1, 128))],
        out_specs=_blk_spec(),
        out_shape=jax.ShapeDtypeStruct((R, 128), f32),
    )(agg1a.reshape(R, 128), agg1b.reshape(R, 128), u1a_b, u1b_b, dis_b,
      M2a, M2b, b1a_t, b1b_t)

    agg2a, agg2b = _make_agg_kernel(False)(
        ei3, u2_b.reshape(N, F), u2_b.reshape(N, F), zeros16)

    out_b = pl.pallas_call(
        _tc_c_body,
        grid=(grid,),
        in_specs=[_blk_spec(), _blk_spec(), _blk_spec(), _blk_spec(),
                  _full_spec((128, 16)), _full_spec((1, 128)),
                  _full_spec((1, 16))],
        out_specs=_blk_spec(16),
        out_shape=jax.ShapeDtypeStruct((R, 16), f32),
    )(agg2a.reshape(R, 128), agg2b.reshape(R, 128), u2_b, dis_b,
      Mfc, b2_t, bfc_t)

    return out_b.reshape(N, 2)


# speedup vs baseline: 96.4307x; 1.0024x over previous
import functools

import jax
import jax.numpy as jnp
from jax import lax
from jax.experimental import pallas as pl
from jax.experimental.pallas import tpu as pltpu
from jax.experimental.pallas import tpu_sc as plsc

N = 100000
E = 6400000
NC = 2
NS = 16
IDXW = 128
G = E // IDXW
K = 5
CB = K * IDXW
F = 16
ZC = 4000
NZC = N // ZC
R = N // 8

C1_MAIN = 624
C2_MAIN = 312
TAIL_G0 = 49920


def _sc_mesh():
    return plsc.VectorSubcoreMesh(core_axis_name="c", subcore_axis_name="s")


def _acc_chunks(sid, fn):
    fn(sid * ZC)

    @pl.when(sid < NZC - NS)
    def _():
        fn((sid + NS) * ZC)


def _make_agg_kernel(feature_split):
    main_cpt = C1_MAIN if feature_split else C2_MAIN

    @functools.partial(
        pl.kernel,
        out_type=[
            jax.ShapeDtypeStruct((N, F), jnp.float32),
            jax.ShapeDtypeStruct((N, F), jnp.float32),
        ],
        mesh=_sc_mesh(),
        scratch_types=[
            pltpu.VMEM((4, K, 2, IDXW), jnp.int32),
            pltpu.VMEM((2, CB, F), jnp.float32),
            pltpu.VMEM_SHARED((N, F), jnp.float32),
            pltpu.SemaphoreType.DMA,
            pltpu.SemaphoreType.DMA,
            pltpu.SemaphoreType.DMA,
        ],
        compiler_params=pltpu.CompilerParams(use_tc_tiling_on_sc=False),
    )
    def agg(ei3, ta, tb, zeros, outa, outb,
            exv, rows, acc, isem, gsem, ssem):
        cid = lax.axis_index("c")
        sid = lax.axis_index("s")

        _acc_chunks(sid, lambda r0: pltpu.sync_copy(
            zeros.at[pl.ds(r0, ZC)], acc.at[pl.ds(r0, ZC)]))
        plsc.subcore_barrier()

        def run(table):
            if feature_split:
                base_g = sid * (C1_MAIN * K)
                tail_g = TAIL_G0 + sid * K
                tail_on = sid >= 0
            else:
                wid = cid * NS + sid
                base_g = wid * (C2_MAIN * K)
                tail_g = TAIL_G0 + wid * K
                tail_on = wid < NS

            def load(i, slot):
                pltpu.async_copy(
                    ei3.at[pl.ds(base_g + i * K, K)], exv.at[slot], isem)

            def drain_load(slot):
                pltpu.make_async_copy(
                    ei3.at[pl.ds(0, K)], exv.at[slot], isem).wait()

            def drain_scatters():
                pltpu.make_async_copy(
                    zeros.at[pl.ds(0, CB)], rows.at[0], ssem).wait()

            load(0, 0)

            def chunk4(i4, carry):
                for b in range(4):
                    i = i4 * 4 + b
                    rs = b % 2

                    @pl.when(i >= 2)
                    def _():
                        drain_scatters()

                    drain_load(b)
                    gds = [
                        pltpu.async_copy(
                            table.at[exv.at[b, j, 0]],
                            rows.at[rs, pl.ds(j * IDXW, IDXW)], gsem)
                        for j in range(K)
                    ]

                    @pl.when(i + 1 < main_cpt)
                    def _():
                        load(i + 1, (b + 1) % 4)

                    for d in gds:
                        d.wait()
                    for j in range(K):
                        pltpu.async_copy(
                            rows.at[rs, pl.ds(j * IDXW, IDXW)],
                            acc.at[exv.at[b, j, 1]], ssem, add=True)
                return carry
            lax.fori_loop(0, main_cpt // 4, chunk4, 0)
            drain_scatters()
            drain_scatters()

            @pl.when(tail_on)
            def _():
                pltpu.sync_copy(ei3.at[pl.ds(tail_g, K)], exv.at[0])
                gds = [
                    pltpu.async_copy(
                        table.at[exv.at[0, j, 0]],
                        rows.at[0, pl.ds(j * IDXW, IDXW)], gsem)
                    for j in range(K)
                ]
                for d in gds:
                    d.wait()
                sds = [
                    pltpu.async_copy(
                        rows.at[0, pl.ds(j * IDXW, IDXW)],
                        acc.at[exv.at[0, j, 1]], ssem, add=True)
                    for j in range(K)
                ]
                for d in sds:
                    d.wait()

        @pl.when(cid == 0)
        def _():
            run(ta)

        @pl.when(cid == 1)
        def _():
            run(tb)

        plsc.subcore_barrier()

        @pl.when(cid == 0)
        def _():
            _acc_chunks(sid, lambda r0: pltpu.sync_copy(
                acc.at[pl.ds(r0, ZC)], outa.at[pl.ds(r0, ZC)]))

        @pl.when(cid == 1)
        def _():
            _acc_chunks(sid, lambda r0: pltpu.sync_copy(
                acc.at[pl.ds(r0, ZC)], outb.at[pl.ds(r0, ZC)]))

    return agg


def _make_deg_kernel():
    @functools.partial(
        pl.kernel,
        out_type=[
            jax.ShapeDtypeStruct((N, F), jnp.float32),
            jax.ShapeDtypeStruct((N, F), jnp.float32),
        ],
        mesh=_sc_mesh(),
        scratch_types=[
            pltpu.VMEM((4, K, IDXW), jnp.int32),
            pltpu.VMEM((IDXW, F), jnp.float32),
            pltpu.VMEM_SHARED((N, F), jnp.float32),
            pltpu.SemaphoreType.DMA,
            pltpu.SemaphoreType.DMA,
        ],
        compiler_params=pltpu.CompilerParams(use_tc_tiling_on_sc=False),
    )
    def deg(ei3, ones, zeros, outa, outb, dstv, ones_v, acc, isem, ssem):
        cid = lax.axis_index("c")
        sid = lax.axis_index("s")
        wid = cid * NS + sid
        base_g = wid * (C2_MAIN * K)

        pltpu.sync_copy(ones, ones_v)
        _acc_chunks(sid, lambda r0: pltpu.sync_copy(
            zeros.at[pl.ds(r0, ZC)], acc.at[pl.ds(r0, ZC)]))
        plsc.subcore_barrier()

        def load(i, dslot):
            pltpu.async_copy(
                ei3.at[pl.ds(base_g + i * K, K), 1], dstv.at[dslot], isem)

        def drain_load(dslot):
            pltpu.make_async_copy(
                ei3.at[pl.ds(0, K), 1], dstv.at[dslot], isem).wait()

        def drain_scatters():
            for _ in range(K):
                pltpu.make_async_copy(
                    zeros.at[pl.ds(0, IDXW)], ones_v, ssem).wait()

        load(0, 0)

        def chunk4(i4, carry):
            for b in range(4):
                i = i4 * 4 + b

                @pl.when(i >= 2)
                def _():
                    drain_scatters()

                drain_load(b)

                @pl.when(i + 1 < C2_MAIN)
                def _():
                    load(i + 1, (b + 1) % 4)

                for j in range(K):
                    pltpu.async_copy(
                        ones_v, acc.at[dstv.at[b, j]], ssem, add=True)
            return carry
        lax.fori_loop(0, C2_MAIN // 4, chunk4, 0)
        drain_scatters()
        drain_scatters()

        @pl.when(wid < NS)
        def _():
            pltpu.sync_copy(
                ei3.at[pl.ds(TAIL_G0 + wid * K, K), 1], dstv.at[0])
            sds = [
                pltpu.async_copy(ones_v, acc.at[dstv.at[0, j]], ssem, add=True)
                for j in range(K)
            ]
            for d in sds:
                d.wait()

        plsc.subcore_barrier()

        @pl.when(cid == 0)
        def _():
            _acc_chunks(sid, lambda r0: pltpu.sync_copy(
                acc.at[pl.ds(r0, ZC)], outa.at[pl.ds(r0, ZC)]))

        @pl.when(cid == 1)
        def _():
            _acc_chunks(sid, lambda r0: pltpu.sync_copy(
                acc.at[pl.ds(r0, ZC)], outb.at[pl.ds(r0, ZC)]))

    return deg


_BR = 1256


def _blk_spec(w=128):
    return pl.BlockSpec((_BR, w), lambda i: (i, 0))


def _full_spec(shape):
    return pl.BlockSpec(shape, lambda i: (0, 0))


def _tc_a_body(x_r, m1a_r, m1b_r, da_r, db_r, u1a_r, u1b_r, dis_r):
    dis = lax.rsqrt(da_r[...] + db_r[...] + 1.0)
    dis_r[...] = dis
    x = x_r[...]
    u1a_r[...] = jnp.dot(x, m1a_r[...], preferred_element_type=jnp.float32) * dis
    u1b_r[...] = jnp.dot(x, m1b_r[...], preferred_element_type=jnp.float32) * dis


def _tc_b_body(aa_r, ab_r, ua_r, ub_r, dis_r, m2a_r, m2b_r, b1a_r, b1b_r, u2_r):
    dis = dis_r[...]
    ha = jnp.maximum((aa_r[...] + ua_r[...]) * dis + b1a_r[...], 0.0)
    hb = jnp.maximum((ab_r[...] + ub_r[...]) * dis + b1b_r[...], 0.0)
    t = (jnp.dot(ha, m2a_r[...], preferred_element_type=jnp.float32)
         + jnp.dot(hb, m2b_r[...], preferred_element_type=jnp.float32))
    u2_r[...] = t * dis


def _tc_c_body(aa_r, ab_r, u2_r, dis_r, mfc_r, b2_r, bfc_r, out_r):
    h = jnp.maximum((aa_r[...] + ab_r[...] + u2_r[...]) * dis_r[...]
                    + b2_r[...], 0.0)
    out_r[...] = (jnp.dot(h, mfc_r[...], preferred_element_type=jnp.float32)
                  + bfc_r[...])


def kernel(x, edge_index, W1, b1, W2, b2, Wfc, bfc):
    f32 = jnp.float32
    ei3 = jnp.transpose(edge_index.reshape(2, G, IDXW), (1, 0, 2))
    xb = jnp.pad(x, ((0, 0), (0, 1))).reshape(R, 64)
    eye8 = jnp.eye(8, dtype=f32)
    W1p = jnp.pad(W1, ((0, 1), (0, 0)))
    M1a = jnp.kron(eye8, W1p[:, :16])
    M1b = jnp.kron(eye8, W1p[:, 16:])
    M2a = jnp.kron(eye8, W2[:16])
    M2b = jnp.kron(eye8, W2[16:])
    Mfc = jnp.kron(eye8, Wfc)
    b1a_t = jnp.tile(b1[:16], 8).reshape(1, 128)
    b1b_t = jnp.tile(b1[16:], 8).reshape(1, 128)
    b2_t = jnp.tile(b2, 8).reshape(1, 128)
    bfc_t = jnp.tile(bfc, 8).reshape(1, 16)
    zeros16 = jnp.zeros((N, F), f32)
    ones = jnp.ones((IDXW, F), f32)

    dega, degb = _make_deg_kernel()(ei3, ones, zeros16)

    grid = (R + _BR - 1) // _BR
    u1a_b, u1b_b, dis_b = pl.pallas_call(
        _tc_a_body,
        grid=(grid,),
        in_specs=[_blk_spec(64), _full_spec((64, 128)), _full_spec((64, 128)),
                  _blk_spec(), _blk_spec()],
        out_specs=[_blk_spec(), _blk_spec(), _blk_spec()],
        out_shape=[jax.ShapeDtypeStruct((R, 128), f32)] * 3,
    )(xb, M1a, M1b, dega.reshape(R, 128), degb.reshape(R, 128))

    agg1a, agg1b = _make_agg_kernel(True)(
        ei3, u1a_b.reshape(N, F), u1b_b.reshape(N, F), zeros16)

    u2_b = pl.pallas_call(
        _tc_b_body,
        grid=(grid,),
        in_specs=[_blk_spec(), _blk_spec(), _blk_spec(), _blk_spec(),
                  _blk_spec(), _full_spec((128, 128)), _full_spec((128, 128)),
                  _full_spec((1, 128)), _full_spec((1, 128))],
        out_specs=_blk_spec(),
        out_shape=jax.ShapeDtypeStruct((R, 128), f32),
    )(agg1a.reshape(R, 128), agg1b.reshape(R, 128), u1a_b, u1b_b, dis_b,
      M2a, M2b, b1a_t, b1b_t)

    agg2a, agg2b = _make_agg_kernel(False)(
        ei3, u2_b.reshape(N, F), u2_b.reshape(N, F), zeros16)

    out_b = pl.pallas_call(
        _tc_c_body,
        grid=(grid,),
        in_specs=[_blk_spec(), _blk_spec(), _blk_spec(), _blk_spec(),
                  _full_spec((128, 16)), _full_spec((1, 128)),
                  _full_spec((1, 16))],
        out_specs=_blk_spec(16),
        out_shape=jax.ShapeDtypeStruct((R, 16), f32),
    )(agg2a.reshape(R, 128), agg2b.reshape(R, 128), u2_b, dis_b,
      Mfc, b2_t, bfc_t)

    return out_b.reshape(N, 2)
